# Initial kernel scaffold; baseline (speedup 1.0000x reference)
import jax
import jax.numpy as jnp
from jax.experimental import pallas as pl


def kernel(x):
    raise NotImplementedError("write your pallas kernel here")


# speedup vs baseline: 19.3236x; 19.3236x over previous
import functools

import jax
import jax.numpy as jnp
from jax import lax
from jax.experimental import pallas as pl
from jax.experimental.pallas import tpu as pltpu
from jax.experimental.pallas import tpu_sc as plsc

B = 128
N = 32768
K_RANK = 1639
NWORKERS = 32
ROWS_PER_W = B // NWORKERS
CHUNKS = N // 16
NBINS = 2048
INT_MIN_I32 = jnp.int32(-(2**31))
INT_MAX_I32 = jnp.int32(2**31 - 1)


def _desc_key(u):
    return u ^ (INT_MIN_I32 | ~(u >> 31))


def _inv_desc_key(kd):
    return jnp.where(kd >= 0, kd ^ INT_MIN_I32, ~kd)


_mesh = plsc.VectorSubcoreMesh(core_axis_name="c", subcore_axis_name="s")


@functools.partial(
    pl.kernel,
    out_type=jax.ShapeDtypeStruct((B, N), jnp.float32),
    mesh=_mesh,
    compiler_params=pltpu.CompilerParams(needs_layout_passes=False),
    scratch_types=[
        pltpu.VMEM((N,), jnp.float32),
        pltpu.VMEM((N,), jnp.int32),
        pltpu.VMEM((NBINS,), jnp.int32),
    ],
)
def _kwta_sc(x_hbm, out_hbm, row_f, row_kd, hist):
    wid = lax.axis_index("s") * 2 + lax.axis_index("c")
    zeros16 = jnp.zeros((16,), jnp.int32)
    ones16 = jnp.ones((16,), jnp.int32)

    def zero_hist(nbins):
        @plsc.parallel_loop(0, nbins // 16, 1, unroll=8)
        def _(c):
            hist[pl.ds(c * 16, 16)] = zeros16

    def scan_hist(nbins, r):
        def body(c, carry):
            csum, nlt, before = carry
            v = hist[pl.ds(c * 16, 16)]
            cum = csum + plsc.cumsum(v)
            lt = cum < r
            nlt = nlt + jnp.sum(lt.astype(jnp.int32))
            before = jnp.maximum(before, jnp.max(jnp.where(lt, cum, 0)))
            csum = jnp.max(cum)
            return csum, nlt, before

        z = jnp.int32(0)
        _, nlt, before = lax.fori_loop(0, nbins // 16, body, (z, z, z))
        return nlt, before

    def do_row(i, _):
        row = wid * ROWS_PER_W + i
        pltpu.sync_copy(x_hbm.at[row], row_f)

        zero_hist(NBINS)

        @plsc.parallel_loop(0, CHUNKS, 1, unroll=4)
        def _(c):
            xv = row_f[pl.ds(c * 16, 16)]
            u = lax.bitcast_convert_type(xv, jnp.int32)
            kd = _desc_key(u)
            row_kd[pl.ds(c * 16, 16)] = kd
            ku = kd ^ INT_MIN_I32
            bins = lax.shift_right_logical(ku, 21)
            plsc.addupdate_scatter(hist, [bins], ones16)

        r1 = jnp.int32(K_RANK)
        b1, before1 = scan_hist(NBINS, r1)
        r2 = r1 - before1

        zero_hist(NBINS)

        @plsc.parallel_loop(0, CHUNKS, 1, unroll=4)
        def _(c):
            kd = row_kd[pl.ds(c * 16, 16)]
            ku = kd ^ INT_MIN_I32
            m = lax.shift_right_logical(ku, 21) == b1
            bins = lax.shift_right_logical(ku, 10) & 0x7FF
            plsc.addupdate_scatter(hist, [bins], ones16, mask=m)

        b2, before2 = scan_hist(NBINS, r2)
        r3 = r2 - before2
        p2 = (b1 << 11) | b2

        zero_hist(1024)

        @plsc.parallel_loop(0, CHUNKS, 1, unroll=4)
        def _(c):
            kd = row_kd[pl.ds(c * 16, 16)]
            ku = kd ^ INT_MIN_I32
            m = lax.shift_right_logical(ku, 10) == p2
            bins = ku & 0x3FF
            plsc.addupdate_scatter(hist, [bins], ones16, mask=m)

        b3, _before3 = scan_hist(1024, r3)
        k1_kd = (((p2 << 10) | b3) ^ INT_MIN_I32).astype(jnp.int32)

        maxs16 = jnp.full((16,), INT_MAX_I32, jnp.int32)

        @plsc.parallel_loop(0, CHUNKS, 1, unroll=4, carry=(zeros16, maxs16))
        def p4(c, carry):
            cnt, mn = carry
            kd = row_kd[pl.ds(c * 16, 16)]
            le = kd <= k1_kd
            cnt = cnt + le.astype(jnp.int32)
            mn = jnp.minimum(mn, jnp.where(le, INT_MAX_I32, kd))
            return cnt, mn

        cnt, mn = p4
        c_le = jnp.sum(cnt)
        k2_kd = jnp.where(c_le >= K_RANK + 1, k1_kd, jnp.min(mn))

        k1v = jnp.full((16,), k1_kd, jnp.int32)
        k2v = jnp.full((16,), k2_kd, jnp.int32)
        va = lax.bitcast_convert_type(_inv_desc_key(k1v), jnp.float32)
        vb = lax.bitcast_convert_type(_inv_desc_key(k2v), jnp.float32)
        t = (va + vb) * jnp.float32(0.5)
        t = jnp.where(t == 0.0, jnp.float32(0.0), t)
        t_kd = _desc_key(lax.bitcast_convert_type(t, jnp.int32))

        @plsc.parallel_loop(0, CHUNKS, 1, unroll=4)
        def _(c):
            kd = row_kd[pl.ds(c * 16, 16)]
            row_f[pl.ds(c * 16, 16)] = jnp.where(
                kd < t_kd, jnp.float32(1.0), jnp.float32(0.0)
            )

        pltpu.sync_copy(row_f, out_hbm.at[row])
        return 0

    lax.fori_loop(0, ROWS_PER_W, do_row, 0)


def kernel(x):
    return _kwta_sc(x)
